# baseline (speedup 1.0000x reference)
import functools

import jax
import jax.numpy as jnp
from jax import lax
from jax.experimental import pallas as pl
from jax.experimental.pallas import tpu as pltpu
from jax.experimental.pallas import tpu_sc as plsc

_N = 10000
_E = 320000
_D = 128
_H = 128
_O = 32

_NC = 2
_NS = 16
_CHUNK = 128
_NSL = 64
_NCHUNK_S = 40
_EPT_S = _NCHUNK_S * _CHUNK
_EPAD = _NSL * _EPT_S
_FSL = 42
_NPAD = 10112
_NROWCHUNKS = _NPAD // _CHUNK
_DUMP = _NPAD - 1

_mesh = plsc.VectorSubcoreMesh(
    core_axis_name="c", subcore_axis_name="s", num_cores=_NC, num_subcores=_NS
)


def _row_chunks_for(sid):
    return (_NROWCHUNKS - sid + _NS - 1) // _NS


_EB = 8192
_NEB = _EPAD // _EB


def _deg_tc_body(dst_ref, out_ref):
    i = pl.program_id(0)
    d = dst_ref[0]
    hi = d >> 7
    lo = d & 127
    io = lax.broadcasted_iota(jnp.int32, (_EB, 128), 1)
    hi_oh = jnp.where(hi == io, 1.0, 0.0).astype(jnp.bfloat16)
    lo_oh = jnp.where(lo == io, 1.0, 0.0).astype(jnp.bfloat16)
    prod = lax.dot_general(
        hi_oh, lo_oh, (((0,), (0,)), ((), ())), preferred_element_type=jnp.float32
    )

    @pl.when(i == 0)
    def _():
        out_ref[...] = jnp.zeros_like(out_ref)

    out_ref[...] += prod


_deg_tc = pl.pallas_call(
    _deg_tc_body,
    grid=(_NEB,),
    in_specs=[pl.BlockSpec((1, _EB, 1), lambda i: (i, 0, 0))],
    out_specs=pl.BlockSpec((128, 128), lambda i: (0, 0)),
    out_shape=jax.ShapeDtypeStruct((128, 128), jnp.float32),
)


@functools.partial(
    pl.kernel,
    out_type=jax.ShapeDtypeStruct((_NC, _NPAD, _H), jnp.float32),
    mesh=_mesh,
    scratch_types=[
        pltpu.VMEM((_EPT_S,), jnp.int32),
        [pltpu.VMEM((_CHUNK,), jnp.int32)] * 2,
        [pltpu.VMEM((_CHUNK, _H), jnp.float32)] * 2,
        pltpu.VMEM_SHARED((_NPAD, _H), jnp.float32),
        [pltpu.SemaphoreType.DMA] * 2,
        [pltpu.SemaphoreType.DMA] * 2,
    ],
)
def _msg_kernel(hs_hbm, src_hbm, dst_hbm, p_out, srcv, dstc, rows, aggsh, gsem, isem):
    cid = lax.axis_index("c")
    sid = lax.axis_index("s")
    zeros16 = jnp.zeros((16,), jnp.float32)

    def zero_rows(i, _):
        def zcol(j, _):
            rows[0][i, pl.ds(j * 16, 16)] = zeros16
            return 0

        lax.fori_loop(0, _H // 16, zcol, 0)
        return 0

    lax.fori_loop(0, _CHUNK, zero_rows, 0)

    nz = _row_chunks_for(sid)

    def zero_shared(k, _):
        c = sid + k * _NS
        pltpu.sync_copy(rows[0], aggsh.at[pl.ds(c * _CHUNK, _CHUNK)])
        return 0

    lax.fori_loop(0, nz, zero_shared, 0)
    plsc.subcore_barrier()

    n_my = jnp.where(
        cid == 0,
        (_FSL - sid + _NS - 1) // _NS,
        ((_NSL - _FSL) - sid + _NS - 1) // _NS,
    )
    base = jnp.where(cid == 0, sid, _FSL + sid)

    def gather(s, c, b):
        pltpu.async_copy(
            hs_hbm.at[srcv.at[pl.ds(c * _CHUNK, _CHUNK)]], rows[b], gsem[b]
        )
        pltpu.async_copy(dst_hbm.at[s, c], dstc[b], isem[b])

    def wait_gather(b):
        pltpu.make_async_copy(hs_hbm.at[pl.ds(0, _CHUNK)], rows[b], gsem[b]).wait()
        pltpu.make_async_copy(dst_hbm.at[0, 0], dstc[b], isem[b]).wait()

    def do_slice(j, _):
        s = base + j * _NS
        pltpu.sync_copy(src_hbm.at[s], srcv)
        gather(s, 0, 0)

        def body(it, _):
            c0 = 2 * it
            gather(s, c0 + 1, 1)
            wait_gather(0)
            pltpu.sync_copy(rows[0], aggsh.at[dstc[0]], add=True)

            @pl.when(c0 + 2 < _NCHUNK_S)
            def _():
                gather(s, c0 + 2, 0)

            wait_gather(1)
            pltpu.sync_copy(rows[1], aggsh.at[dstc[1]], add=True)
            return 0

        lax.fori_loop(0, _NCHUNK_S // 2, body, 0)
        return 0

    lax.fori_loop(0, n_my, do_slice, 0)
    plsc.subcore_barrier()

    def copy_out(k, _):
        c = sid + k * _NS
        sl = pl.ds(c * _CHUNK, _CHUNK)
        pltpu.sync_copy(aggsh.at[sl], p_out.at[cid, sl])
        return 0

    lax.fori_loop(0, nz, copy_out, 0)


def _pre_body(deg_ref, x_ref, w1_ref, hs_ref, dinv_ref):
    deg = deg_ref[...] + 1.0
    di = lax.rsqrt(deg)
    dinv_ref[...] = di
    hs_ref[...] = jnp.dot(x_ref[...], w1_ref[...], preferred_element_type=jnp.float32) * di


_pre_call = pl.pallas_call(
    _pre_body,
    out_shape=(
        jax.ShapeDtypeStruct((_N, _H), jnp.float32),
        jax.ShapeDtypeStruct((_N, 1), jnp.float32),
    ),
)


def _mid_body(p_ref, hs_ref, dinv_ref, b1_ref, g_ref, be_ref, w2_ref, hs2_ref):
    di = dinv_ref[...]
    p = p_ref[0, :_N, :] + p_ref[1, :_N, :]
    t = di * (p + hs_ref[...]) + b1_ref[...]
    h = jnp.where(t > 0, t, jnp.exp(t) - 1.0)
    mean = jnp.mean(h, axis=0, keepdims=True)
    var = jnp.mean((h - mean) * (h - mean), axis=0, keepdims=True)
    h = (h - mean) * lax.rsqrt(var + 1e-5) * g_ref[...] + be_ref[...]
    hs2_ref[...] = jnp.dot(h, w2_ref[...], preferred_element_type=jnp.float32) * di


_mid_call = pl.pallas_call(
    _mid_body,
    out_shape=jax.ShapeDtypeStruct((_N, _H), jnp.float32),
)


def _post_body(p_ref, hs_ref, dinv_ref, b2_ref, wc_ref, bc_ref, out_ref):
    di = dinv_ref[...]
    p = p_ref[0, :_N, :] + p_ref[1, :_N, :]
    t = di * (p + hs_ref[...]) + b2_ref[...]
    h = jnp.where(t > 0, t, jnp.exp(t) - 1.0)
    logits = jnp.dot(h, wc_ref[...], preferred_element_type=jnp.float32) + bc_ref[...]
    m = jnp.max(logits, axis=1, keepdims=True)
    lse = m + jnp.log(jnp.sum(jnp.exp(logits - m), axis=1, keepdims=True))
    out_ref[...] = logits - lse


_post_call = pl.pallas_call(
    _post_body,
    out_shape=jax.ShapeDtypeStruct((_N, _O), jnp.float32),
)


@jax.jit
def kernel(x, edge_index, W1, b1, W2, b2, gamma, beta, Wc, bc):
    src = edge_index[0].astype(jnp.int32)
    dst = edge_index[1].astype(jnp.int32)
    npad = _EPAD - _E
    srcp = jnp.concatenate([src, jnp.zeros((npad,), jnp.int32)]).reshape(_NSL, _EPT_S)
    dstp = jnp.concatenate([dst, jnp.full((npad,), _DUMP, jnp.int32)]).reshape(
        _NSL, _NCHUNK_S, _CHUNK
    )

    deg2d = _deg_tc(dstp.reshape(_NEB, _EB, 1))
    deg = deg2d.reshape(-1)[:_N].reshape(_N, 1)
    hs1, dinv = _pre_call(deg, x, W1)
    p1 = _msg_kernel(hs1, srcp, dstp)
    hs2 = _mid_call(
        p1, hs1, dinv, b1.reshape(1, _H), gamma.reshape(1, _H), beta.reshape(1, _H), W2
    )
    p2 = _msg_kernel(hs2, srcp, dstp)
    return _post_call(p2, hs2, dinv, b2.reshape(1, _H), Wc, bc.reshape(1, _O))


# speedup vs baseline: 1.1816x; 1.1816x over previous
import functools

import jax
import jax.numpy as jnp
from jax import lax
from jax.experimental import pallas as pl
from jax.experimental.pallas import tpu as pltpu
from jax.experimental.pallas import tpu_sc as plsc

_N = 10000
_E = 320000
_D = 128
_H = 128
_O = 32

_NC = 2
_NS = 16
_CHUNK = 128
_NSL = 64
_NCHUNK_S = 40
_EPT_S = _NCHUNK_S * _CHUNK
_EPAD = _NSL * _EPT_S
_FSL = 32
_NPAD = 10112
_NROWCHUNKS = _NPAD // _CHUNK
_DUMP = _NPAD - 1

_mesh = plsc.VectorSubcoreMesh(
    core_axis_name="c", subcore_axis_name="s", num_cores=_NC, num_subcores=_NS
)


def _row_chunks_for(sid):
    return (_NROWCHUNKS - sid + _NS - 1) // _NS


_EB = 8192
_NEB = _EPAD // _EB


def _deg_tc_body(dst_ref, out_ref):
    i = pl.program_id(0)
    d = dst_ref[0]
    hi = d >> 7
    lo = d & 127
    io = lax.broadcasted_iota(jnp.int32, (_EB, 128), 1)
    hi_oh = jnp.where(hi == io, 1.0, 0.0).astype(jnp.bfloat16)
    lo_oh = jnp.where(lo == io, 1.0, 0.0).astype(jnp.bfloat16)
    prod = lax.dot_general(
        hi_oh, lo_oh, (((0,), (0,)), ((), ())), preferred_element_type=jnp.float32
    )

    @pl.when(i == 0)
    def _():
        out_ref[...] = jnp.zeros_like(out_ref)

    out_ref[...] += prod


_deg_tc = pl.pallas_call(
    _deg_tc_body,
    grid=(_NEB,),
    in_specs=[pl.BlockSpec((1, _EB, 1), lambda i: (i, 0, 0))],
    out_specs=pl.BlockSpec((128, 128), lambda i: (0, 0)),
    out_shape=jax.ShapeDtypeStruct((128, 128), jnp.float32),
)


@functools.partial(
    pl.kernel,
    out_type=jax.ShapeDtypeStruct((_NC, _NPAD, _H), jnp.float32),
    mesh=_mesh,
    scratch_types=[
        pltpu.VMEM((_EPT_S,), jnp.int32),
        pltpu.VMEM((_NCHUNK_S, _CHUNK), jnp.int32),
        [pltpu.VMEM((_CHUNK, _H), jnp.float32)] * 2,
        pltpu.VMEM_SHARED((_NPAD, _H), jnp.float32),
        [pltpu.SemaphoreType.DMA] * 2,
        pltpu.SemaphoreType.DMA,
    ],
)
def _msg_kernel(hs_hbm, src_hbm, dst_hbm, p_out, srcv, dstv, rows, aggsh, gsem, isem):
    cid = lax.axis_index("c")
    sid = lax.axis_index("s")
    zeros16 = jnp.zeros((16,), jnp.float32)

    def zero_rows(i, _):
        def zcol(j, _):
            rows[0][i, pl.ds(j * 16, 16)] = zeros16
            return 0

        lax.fori_loop(0, _H // 16, zcol, 0)
        return 0

    lax.fori_loop(0, _CHUNK, zero_rows, 0)

    nz = _row_chunks_for(sid)

    def zero_shared(k, _):
        c = sid + k * _NS
        pltpu.sync_copy(rows[0], aggsh.at[pl.ds(c * _CHUNK, _CHUNK)])
        return 0

    lax.fori_loop(0, nz, zero_shared, 0)
    plsc.subcore_barrier()

    n_my = jnp.where(
        cid == 0,
        (_FSL - sid + _NS - 1) // _NS,
        ((_NSL - _FSL) - sid + _NS - 1) // _NS,
    )
    base = jnp.where(cid == 0, sid, _FSL + sid)

    def gather(c, b):
        pltpu.async_copy(
            hs_hbm.at[srcv.at[pl.ds(c * _CHUNK, _CHUNK)]], rows[b], gsem[b]
        )

    def wait_gather(b):
        pltpu.make_async_copy(hs_hbm.at[pl.ds(0, _CHUNK)], rows[b], gsem[b]).wait()

    def do_slice(j, _):
        s = base + j * _NS
        pltpu.async_copy(src_hbm.at[s], srcv, isem)
        pltpu.async_copy(dst_hbm.at[s], dstv, isem)
        pltpu.make_async_copy(src_hbm.at[0], srcv, isem).wait()
        pltpu.make_async_copy(dst_hbm.at[0], dstv, isem).wait()
        gather(0, 0)

        def body(it, _):
            c0 = 2 * it
            gather(c0 + 1, 1)
            wait_gather(0)
            pltpu.sync_copy(rows[0], aggsh.at[dstv.at[c0]], add=True)

            @pl.when(c0 + 2 < _NCHUNK_S)
            def _():
                gather(c0 + 2, 0)

            wait_gather(1)
            pltpu.sync_copy(rows[1], aggsh.at[dstv.at[c0 + 1]], add=True)
            return 0

        lax.fori_loop(0, _NCHUNK_S // 2, body, 0)
        return 0

    lax.fori_loop(0, n_my, do_slice, 0)
    plsc.subcore_barrier()

    def copy_out(k, _):
        c = sid + k * _NS
        sl = pl.ds(c * _CHUNK, _CHUNK)
        pltpu.sync_copy(aggsh.at[sl], p_out.at[cid, sl])
        return 0

    lax.fori_loop(0, nz, copy_out, 0)


def _pre_body(deg_ref, x_ref, w1_ref, hs_ref, dinv_ref):
    deg = deg_ref[...] + 1.0
    di = lax.rsqrt(deg)
    dinv_ref[...] = di
    hs_ref[...] = jnp.dot(x_ref[...], w1_ref[...], preferred_element_type=jnp.float32) * di


_pre_call = pl.pallas_call(
    _pre_body,
    out_shape=(
        jax.ShapeDtypeStruct((_N, _H), jnp.float32),
        jax.ShapeDtypeStruct((_N, 1), jnp.float32),
    ),
)


def _mid_body(p_ref, hs_ref, dinv_ref, b1_ref, g_ref, be_ref, w2_ref, hs2_ref):
    di = dinv_ref[...]
    p = p_ref[0, :_N, :] + p_ref[1, :_N, :]
    t = di * (p + hs_ref[...]) + b1_ref[...]
    h = jnp.where(t > 0, t, jnp.exp(t) - 1.0)
    mean = jnp.mean(h, axis=0, keepdims=True)
    var = jnp.mean((h - mean) * (h - mean), axis=0, keepdims=True)
    h = (h - mean) * lax.rsqrt(var + 1e-5) * g_ref[...] + be_ref[...]
    hs2_ref[...] = jnp.dot(h, w2_ref[...], preferred_element_type=jnp.float32) * di


_mid_call = pl.pallas_call(
    _mid_body,
    out_shape=jax.ShapeDtypeStruct((_N, _H), jnp.float32),
)


def _post_body(p_ref, hs_ref, dinv_ref, b2_ref, wc_ref, bc_ref, out_ref):
    di = dinv_ref[...]
    p = p_ref[0, :_N, :] + p_ref[1, :_N, :]
    t = di * (p + hs_ref[...]) + b2_ref[...]
    h = jnp.where(t > 0, t, jnp.exp(t) - 1.0)
    logits = jnp.dot(h, wc_ref[...], preferred_element_type=jnp.float32) + bc_ref[...]
    m = jnp.max(logits, axis=1, keepdims=True)
    lse = m + jnp.log(jnp.sum(jnp.exp(logits - m), axis=1, keepdims=True))
    out_ref[...] = logits - lse


_post_call = pl.pallas_call(
    _post_body,
    out_shape=jax.ShapeDtypeStruct((_N, _O), jnp.float32),
)


@jax.jit
def kernel(x, edge_index, W1, b1, W2, b2, gamma, beta, Wc, bc):
    src = edge_index[0].astype(jnp.int32)
    dst = edge_index[1].astype(jnp.int32)
    npad = _EPAD - _E
    srcp = jnp.concatenate([src, jnp.zeros((npad,), jnp.int32)]).reshape(_NSL, _EPT_S)
    dstp = jnp.concatenate([dst, jnp.full((npad,), _DUMP, jnp.int32)]).reshape(
        _NSL, _NCHUNK_S, _CHUNK
    )

    deg2d = _deg_tc(dstp.reshape(_NEB, _EB, 1))
    deg = deg2d.reshape(-1)[:_N].reshape(_N, 1)
    hs1, dinv = _pre_call(deg, x, W1)
    p1 = _msg_kernel(hs1, srcp, dstp)
    hs2 = _mid_call(
        p1, hs1, dinv, b1.reshape(1, _H), gamma.reshape(1, _H), beta.reshape(1, _H), W2
    )
    p2 = _msg_kernel(hs2, srcp, dstp)
    return _post_call(p2, hs2, dinv, b2.reshape(1, _H), Wc, bc.reshape(1, _O))


# speedup vs baseline: 1.6376x; 1.3859x over previous
import functools

import jax
import jax.numpy as jnp
from jax import lax
from jax.experimental import pallas as pl
from jax.experimental.pallas import tpu as pltpu
from jax.experimental.pallas import tpu_sc as plsc

_N = 10000
_E = 320000
_D = 128
_H = 128
_O = 32

_NC = 2
_NS = 16
_CHUNK = 128
_NSL = 64
_NCHUNK_S = 40
_EPT_S = _NCHUNK_S * _CHUNK
_EPAD = _NSL * _EPT_S
_FSL = 32
_NPAD = 10112
_NROWCHUNKS = _NPAD // _CHUNK
_DUMP = _NPAD - 1

_mesh = plsc.VectorSubcoreMesh(
    core_axis_name="c", subcore_axis_name="s", num_cores=_NC, num_subcores=_NS
)


def _row_chunks_for(sid):
    return (_NROWCHUNKS - sid + _NS - 1) // _NS


_EB = 8192
_NEB = _EPAD // _EB


def _deg_tc_body(dst_ref, out_ref):
    i = pl.program_id(0)
    d = dst_ref[0]
    hi = d >> 7
    lo = d & 127
    io = lax.broadcasted_iota(jnp.int32, (128, _EB), 0)
    hi_oh = jnp.where(hi == io, 1.0, 0.0).astype(jnp.bfloat16)
    lo_oh = jnp.where(lo == io, 1.0, 0.0).astype(jnp.bfloat16)
    prod = lax.dot_general(
        hi_oh, lo_oh, (((1,), (1,)), ((), ())), preferred_element_type=jnp.float32
    )

    @pl.when(i == 0)
    def _():
        out_ref[...] = jnp.zeros_like(out_ref)

    out_ref[...] += prod


_deg_tc = pl.pallas_call(
    _deg_tc_body,
    grid=(_NEB,),
    in_specs=[pl.BlockSpec((1, 1, _EB), lambda i: (i, 0, 0))],
    out_specs=pl.BlockSpec((128, 128), lambda i: (0, 0)),
    out_shape=jax.ShapeDtypeStruct((128, 128), jnp.float32),
)


@functools.partial(
    pl.kernel,
    out_type=jax.ShapeDtypeStruct((_NC, _NPAD, _H), jnp.float32),
    mesh=_mesh,
    scratch_types=[
        pltpu.VMEM((_EPT_S,), jnp.int32),
        pltpu.VMEM((_NCHUNK_S, _CHUNK), jnp.int32),
        [pltpu.VMEM((_CHUNK, _H), jnp.float32)] * 2,
        pltpu.VMEM_SHARED((_NPAD, _H), jnp.float32),
        [pltpu.SemaphoreType.DMA] * 2,
        pltpu.SemaphoreType.DMA,
    ],
)
def _msg_kernel(hs_hbm, src_hbm, dst_hbm, p_out, srcv, dstv, rows, aggsh, gsem, isem):
    cid = lax.axis_index("c")
    sid = lax.axis_index("s")
    zeros16 = jnp.zeros((16,), jnp.float32)

    def zero_rows(i, _):
        def zcol(j, _):
            rows[0][i, pl.ds(j * 16, 16)] = zeros16
            return 0

        lax.fori_loop(0, _H // 16, zcol, 0)
        return 0

    lax.fori_loop(0, _CHUNK, zero_rows, 0)

    nz = _row_chunks_for(sid)

    def zero_shared(k, _):
        c = sid + k * _NS
        pltpu.sync_copy(rows[0], aggsh.at[pl.ds(c * _CHUNK, _CHUNK)])
        return 0

    lax.fori_loop(0, nz, zero_shared, 0)
    plsc.subcore_barrier()

    n_my = jnp.where(
        cid == 0,
        (_FSL - sid + _NS - 1) // _NS,
        ((_NSL - _FSL) - sid + _NS - 1) // _NS,
    )
    base = jnp.where(cid == 0, sid, _FSL + sid)

    def gather(c, b):
        pltpu.async_copy(
            hs_hbm.at[srcv.at[pl.ds(c * _CHUNK, _CHUNK)]], rows[b], gsem[b]
        )

    def wait_gather(b):
        pltpu.make_async_copy(hs_hbm.at[pl.ds(0, _CHUNK)], rows[b], gsem[b]).wait()

    def do_slice(j, _):
        s = base + j * _NS
        pltpu.async_copy(src_hbm.at[s], srcv, isem)
        pltpu.async_copy(dst_hbm.at[s], dstv, isem)
        pltpu.make_async_copy(src_hbm.at[0], srcv, isem).wait()
        pltpu.make_async_copy(dst_hbm.at[0], dstv, isem).wait()
        gather(0, 0)

        def body(it, _):
            c0 = 2 * it
            gather(c0 + 1, 1)
            wait_gather(0)
            pltpu.sync_copy(rows[0], aggsh.at[dstv.at[c0]], add=True)

            @pl.when(c0 + 2 < _NCHUNK_S)
            def _():
                gather(c0 + 2, 0)

            wait_gather(1)
            pltpu.sync_copy(rows[1], aggsh.at[dstv.at[c0 + 1]], add=True)
            return 0

        lax.fori_loop(0, _NCHUNK_S // 2, body, 0)
        return 0

    lax.fori_loop(0, n_my, do_slice, 0)
    plsc.subcore_barrier()

    def copy_out(k, _):
        c = sid + k * _NS
        sl = pl.ds(c * _CHUNK, _CHUNK)
        pltpu.sync_copy(aggsh.at[sl], p_out.at[cid, sl])
        return 0

    lax.fori_loop(0, nz, copy_out, 0)


def _pre_body(deg_ref, x_ref, w1_ref, hs_ref, dinv_ref):
    deg = deg_ref[...] + 1.0
    di = lax.rsqrt(deg)
    dinv_ref[...] = di
    hs_ref[...] = jnp.dot(x_ref[...], w1_ref[...], preferred_element_type=jnp.float32) * di


_pre_call = pl.pallas_call(
    _pre_body,
    out_shape=(
        jax.ShapeDtypeStruct((_N, _H), jnp.float32),
        jax.ShapeDtypeStruct((_N, 1), jnp.float32),
    ),
)


def _mid_body(p_ref, hs_ref, dinv_ref, b1_ref, g_ref, be_ref, w2_ref, hs2_ref):
    di = dinv_ref[...]
    p = p_ref[0, :_N, :] + p_ref[1, :_N, :]
    t = di * (p + hs_ref[...]) + b1_ref[...]
    h = jnp.where(t > 0, t, jnp.exp(t) - 1.0)
    mean = jnp.mean(h, axis=0, keepdims=True)
    var = jnp.mean((h - mean) * (h - mean), axis=0, keepdims=True)
    h = (h - mean) * lax.rsqrt(var + 1e-5) * g_ref[...] + be_ref[...]
    hs2_ref[...] = jnp.dot(h, w2_ref[...], preferred_element_type=jnp.float32) * di


_mid_call = pl.pallas_call(
    _mid_body,
    out_shape=jax.ShapeDtypeStruct((_N, _H), jnp.float32),
)


def _post_body(p_ref, hs_ref, dinv_ref, b2_ref, wc_ref, bc_ref, out_ref):
    di = dinv_ref[...]
    p = p_ref[0, :_N, :] + p_ref[1, :_N, :]
    t = di * (p + hs_ref[...]) + b2_ref[...]
    h = jnp.where(t > 0, t, jnp.exp(t) - 1.0)
    logits = jnp.dot(h, wc_ref[...], preferred_element_type=jnp.float32) + bc_ref[...]
    m = jnp.max(logits, axis=1, keepdims=True)
    lse = m + jnp.log(jnp.sum(jnp.exp(logits - m), axis=1, keepdims=True))
    out_ref[...] = logits - lse


_post_call = pl.pallas_call(
    _post_body,
    out_shape=jax.ShapeDtypeStruct((_N, _O), jnp.float32),
)


@jax.jit
def kernel(x, edge_index, W1, b1, W2, b2, gamma, beta, Wc, bc):
    src = edge_index[0].astype(jnp.int32)
    dst = edge_index[1].astype(jnp.int32)
    npad = _EPAD - _E
    srcp = jnp.concatenate([src, jnp.zeros((npad,), jnp.int32)]).reshape(_NSL, _EPT_S)
    dstp = jnp.concatenate([dst, jnp.full((npad,), _DUMP, jnp.int32)]).reshape(
        _NSL, _NCHUNK_S, _CHUNK
    )

    deg2d = _deg_tc(dstp.reshape(_NEB, 1, _EB))
    deg = deg2d.reshape(-1)[:_N].reshape(_N, 1)
    hs1, dinv = _pre_call(deg, x, W1)
    p1 = _msg_kernel(hs1, srcp, dstp)
    hs2 = _mid_call(
        p1, hs1, dinv, b1.reshape(1, _H), gamma.reshape(1, _H), beta.reshape(1, _H), W2
    )
    p2 = _msg_kernel(hs2, srcp, dstp)
    return _post_call(p2, hs2, dinv, b2.reshape(1, _H), Wc, bc.reshape(1, _O))
